# baseline (speedup 1.0000x reference)
import jax
import jax.numpy as jnp
from jax import lax
from jax.experimental import pallas as pl
from jax.experimental.pallas import tpu as pltpu
from jax.experimental.pallas import tpu_sc as plsc

VOCAB = 1000000
EMBED_DIM = 64
FEATURE_LEN = 26
BATCH = 4096
HIST = 50
LAYER1 = 128
CLASS_SIZE = 100

NUM_WORKERS = 32
CHUNK = 128
R_TOT = BATCH * FEATURE_LEN
KTILE = FEATURE_LEN // 2

_SC_MESH = plsc.VectorSubcoreMesh(core_axis_name="c", subcore_axis_name="s")


def _sc_pool_body(emb_hbm, idx_hbm, out_hbm, blk, idxT, acc, dsti,
                  sem_g0, sem_g1):
    wid = lax.axis_index("s") * 2 + lax.axis_index("c")
    b0 = wid * CHUNK
    col16 = lax.iota(jnp.int32, 16)
    zeros16 = jnp.zeros((16,), jnp.float32)

    def prep(par, f):
        pltpu.sync_copy(idx_hbm.at[pl.ds(f * BATCH + b0, CHUNK)], blk)

        @pl.loop(0, HIST)
        def _t(h):
            hv = jnp.zeros((16,), jnp.int32) + h
            for k in range(CHUNK // 16):
                vals = plsc.load_gather(blk, [col16 + (k * 16), hv])
                idxT[par, h, pl.ds(k * 16, 16)] = vals

    def zero_acc(par):
        @pl.loop(0, CHUNK)
        def _z(i):
            for j in range(EMBED_DIM // 16):
                acc[par, i, pl.ds(j * 16, 16)] = zeros16

    def fire(par, sem):
        @pl.loop(0, HIST)
        def _g(h):
            pltpu.async_copy(emb_hbm.at[idxT.at[par, h]], acc.at[par], sem,
                             add=True)

    def drain(par, sem):
        @pl.loop(0, HIST)
        def _w(h):
            pltpu.make_async_copy(emb_hbm.at[idxT.at[par, h]], acc.at[par],
                                  sem).wait()

    def scatter(par, f):
        base = 2 * b0 + 8192 * (f // 2) + (f % 2)
        for k in range(CHUNK // 16):
            dsti[pl.ds(k * 16, 16)] = (col16 + (k * 16)) * 2 + base
        pltpu.sync_copy(acc.at[par], out_hbm.at[dsti])

    sems = (sem_g0, sem_g1)
    for par in range(2):
        prep(par, par)
        zero_acc(par)
        fire(par, sems[par])

    @pl.loop(0, FEATURE_LEN // 2)
    def _outer(fo):
        for par in range(2):
            f = fo * 2 + par
            drain(par, sems[par])
            scatter(par, f)
            nxt = f + 2

            @pl.when(nxt < FEATURE_LEN)
            def _p():
                prep(par, nxt)
                zero_acc(par)
                fire(par, sems[par])


@jax.jit
def _sc_pool(emb, idx):
    k = pl.kernel(
        _sc_pool_body,
        out_type=jax.ShapeDtypeStruct((R_TOT, EMBED_DIM), jnp.float32),
        mesh=_SC_MESH,
        scratch_types=[
            pltpu.VMEM((CHUNK, 128), jnp.int32),
            pltpu.VMEM((2, HIST, CHUNK), jnp.int32),
            pltpu.VMEM((2, CHUNK, EMBED_DIM), jnp.float32),
            pltpu.VMEM((CHUNK,), jnp.int32),
            pltpu.SemaphoreType.DMA,
            pltpu.SemaphoreType.DMA,
        ],
        compiler_params=pltpu.CompilerParams(use_tc_tiling_on_sc=False,
                                             needs_layout_passes=False),
    )
    return k(emb, idx)


BB = 256


def _tc_mlp_body(x_ref, w1_ref, b1_ref, w2_ref, b2_ref, o_ref):
    h = jnp.dot(x_ref[0], w1_ref[0], preferred_element_type=jnp.float32)
    for k in range(1, KTILE):
        h += jnp.dot(x_ref[k], w1_ref[k], preferred_element_type=jnp.float32)
    h = h * (1.0 / BATCH) + b1_ref[...]
    h = jnp.maximum(h, 0.0)
    o = jnp.dot(h, w2_ref[...], preferred_element_type=jnp.float32) + b2_ref[...]
    m = jnp.max(o, axis=1, keepdims=True)
    e = jnp.exp(o - m)
    lse = jnp.log(jnp.sum(e, axis=1, keepdims=True)) + m
    o_ref[...] = o - lse


@jax.jit
def _tc_mlp(x3, W13, b1, W2, b2):
    return pl.pallas_call(
        _tc_mlp_body,
        grid=(BATCH // BB,),
        in_specs=[
            pl.BlockSpec((KTILE, BB, LAYER1), lambda i: (0, i, 0)),
            pl.BlockSpec((KTILE, LAYER1, LAYER1), lambda i: (0, 0, 0)),
            pl.BlockSpec((1, LAYER1), lambda i: (0, 0)),
            pl.BlockSpec((LAYER1, CLASS_SIZE), lambda i: (0, 0)),
            pl.BlockSpec((1, CLASS_SIZE), lambda i: (0, 0)),
        ],
        out_specs=pl.BlockSpec((BB, CLASS_SIZE), lambda i: (i, 0)),
        out_shape=jax.ShapeDtypeStruct((BATCH, CLASS_SIZE), jnp.float32),
    )(x3, W13, b1, W2, b2)


def kernel(batch_inputs, emb, W1, b1, W2, b2):
    idx = batch_inputs.astype(jnp.int32)
    idxp = jnp.pad(idx, ((0, 0), (0, 0), (0, 128 - HIST))).reshape(R_TOT, 128)
    pooled = _sc_pool(emb, idxp)
    x3 = pooled.reshape(KTILE, BATCH, LAYER1)
    W13 = W1.reshape(KTILE, LAYER1, LAYER1)
    return _tc_mlp(x3, W13, b1.reshape(1, -1), W2, b2.reshape(1, -1))


# speedup vs baseline: 1.4058x; 1.4058x over previous
import jax
import jax.numpy as jnp
from jax import lax
from jax.experimental import pallas as pl
from jax.experimental.pallas import tpu as pltpu
from jax.experimental.pallas import tpu_sc as plsc

VOCAB = 1000000
EMBED_DIM = 64
FEATURE_LEN = 26
BATCH = 4096
HIST = 50
LAYER1 = 128
CLASS_SIZE = 100

NUM_WORKERS = 32
CHUNK = 128
R_TOT = BATCH * FEATURE_LEN
KTILE = FEATURE_LEN // 2

_SC_MESH = plsc.VectorSubcoreMesh(core_axis_name="c", subcore_axis_name="s")


def _sc_pool_body(emb_hbm, idx_hbm, out_hbm, blk, idxT, acc, dsti,
                  sem_g0, sem_g1):
    wid = lax.axis_index("s") * 2 + lax.axis_index("c")
    b0 = wid * CHUNK
    col16 = lax.iota(jnp.int32, 16)
    zeros16 = jnp.zeros((16,), jnp.float32)

    def prep(par, f):
        pltpu.sync_copy(idx_hbm.at[pl.ds(f * BATCH + b0, CHUNK)], blk)

        @pl.loop(0, HIST)
        def _t(h):
            hv = jnp.zeros((16,), jnp.int32) + h
            for k in range(CHUNK // 16):
                vals = plsc.load_gather(blk, [col16 + (k * 16), hv])
                idxT[par, h, pl.ds(k * 16, 16)] = vals * 2

    def zero_acc(par):
        @pl.loop(0, CHUNK)
        def _z(i):
            for j in range(EMBED_DIM // 16):
                acc[par, i, pl.ds(j * 16, 16)] = zeros16

    def fire(par, sem):
        @pl.loop(0, HIST)
        def _g(h):
            pltpu.async_copy(emb_hbm.at[idxT.at[par, h]], acc.at[par], sem,
                             add=True)

    def drain(par, sem):
        @pl.loop(0, HIST)
        def _w(h):
            pltpu.make_async_copy(emb_hbm.at[idxT.at[par, h]], acc.at[par],
                                  sem).wait()

    def scatter(par, f):
        base = 2 * b0 + 8192 * (f // 2) + (f % 2)
        for k in range(CHUNK // 16):
            dsti[pl.ds(k * 16, 16)] = (col16 + (k * 16)) * 2 + base
        pltpu.sync_copy(acc.at[par], out_hbm.at[dsti])

    sems = (sem_g0, sem_g1)
    for par in range(2):
        prep(par, par)
        zero_acc(par)
        fire(par, sems[par])

    @pl.loop(0, FEATURE_LEN // 2)
    def _outer(fo):
        for par in range(2):
            f = fo * 2 + par
            drain(par, sems[par])
            scatter(par, f)
            nxt = f + 2

            @pl.when(nxt < FEATURE_LEN)
            def _p():
                prep(par, nxt)
                zero_acc(par)
                fire(par, sems[par])


@jax.jit
def _sc_pool(emb2m, idx):
    k = pl.kernel(
        _sc_pool_body,
        out_type=jax.ShapeDtypeStruct((R_TOT, EMBED_DIM), jnp.float32),
        mesh=_SC_MESH,
        scratch_types=[
            pltpu.VMEM((CHUNK, 128), jnp.int32),
            pltpu.VMEM((2, HIST, CHUNK), jnp.int32),
            pltpu.VMEM((2, CHUNK, EMBED_DIM), jnp.float32),
            pltpu.VMEM((CHUNK,), jnp.int32),
            pltpu.SemaphoreType.DMA,
            pltpu.SemaphoreType.DMA,
        ],
        compiler_params=pltpu.CompilerParams(use_tc_tiling_on_sc=False,
                                             needs_layout_passes=False),
    )
    return k(emb2m, idx)


def _padidx_body(x_ref, o_ref):
    o_ref[:, 0:HIST] = x_ref[0].T


@jax.jit
def _pad_idx(idxT):
    return pl.pallas_call(
        _padidx_body,
        grid=(FEATURE_LEN,),
        in_specs=[pl.BlockSpec((1, HIST, BATCH), lambda i: (i, 0, 0))],
        out_specs=pl.BlockSpec((BATCH, 128), lambda i: (i, 0)),
        out_shape=jax.ShapeDtypeStruct((R_TOT, 128), jnp.int32),
    )(idxT)


PT_B = 15872


def _padT_body(x_ref, o_ref):
    o_ref[:, 0:EMBED_DIM] = x_ref[...].T


@jax.jit
def _pad_table(embT):
    return pl.pallas_call(
        _padT_body,
        grid=(pl.cdiv(VOCAB, PT_B),),
        in_specs=[pl.BlockSpec((EMBED_DIM, PT_B), lambda i: (0, i))],
        out_specs=pl.BlockSpec((PT_B, 2 * EMBED_DIM), lambda i: (i, 0)),
        out_shape=jax.ShapeDtypeStruct((VOCAB, 2 * EMBED_DIM), jnp.float32),
    )(embT)


BB = 256


def _tc_mlp_body(x_ref, w1_ref, b1_ref, w2_ref, b2_ref, o_ref):
    h = jnp.dot(x_ref[0], w1_ref[0], preferred_element_type=jnp.float32)
    for k in range(1, KTILE):
        h += jnp.dot(x_ref[k], w1_ref[k], preferred_element_type=jnp.float32)
    h = h * (1.0 / BATCH) + b1_ref[...]
    h = jnp.maximum(h, 0.0)
    o = jnp.dot(h, w2_ref[...], preferred_element_type=jnp.float32) + b2_ref[...]
    m = jnp.max(o, axis=1, keepdims=True)
    e = jnp.exp(o - m)
    lse = jnp.log(jnp.sum(e, axis=1, keepdims=True)) + m
    o_ref[...] = o - lse


@jax.jit
def _tc_mlp(x3, W13, b1, W2, b2):
    return pl.pallas_call(
        _tc_mlp_body,
        grid=(BATCH // BB,),
        in_specs=[
            pl.BlockSpec((KTILE, BB, LAYER1), lambda i: (0, i, 0)),
            pl.BlockSpec((KTILE, LAYER1, LAYER1), lambda i: (0, 0, 0)),
            pl.BlockSpec((1, LAYER1), lambda i: (0, 0)),
            pl.BlockSpec((LAYER1, CLASS_SIZE), lambda i: (0, 0)),
            pl.BlockSpec((1, CLASS_SIZE), lambda i: (0, 0)),
        ],
        out_specs=pl.BlockSpec((BB, CLASS_SIZE), lambda i: (i, 0)),
        out_shape=jax.ShapeDtypeStruct((BATCH, CLASS_SIZE), jnp.float32),
    )(x3, W13, b1, W2, b2)


def kernel(batch_inputs, emb, W1, b1, W2, b2):
    idx = batch_inputs.astype(jnp.int32)
    idxp = _pad_idx(idx.transpose(0, 2, 1))
    emb2m = _pad_table(emb.T).reshape(2 * VOCAB, EMBED_DIM)
    pooled = _sc_pool(emb2m, idxp)
    x3 = pooled.reshape(KTILE, BATCH, LAYER1)
    W13 = W1.reshape(KTILE, LAYER1, LAYER1)
    return _tc_mlp(x3, W13, b1.reshape(1, -1), W2, b2.reshape(1, -1))
